# baseline (speedup 1.0000x reference)
import functools

import jax
import jax.numpy as jnp
from jax.experimental import pallas as pl
from jax.experimental.pallas import tpu as pltpu


def _lstm_body(x_ref, wih0_ref, whh0_ref, b0_ref, wih1_ref, whh1_ref, b1_ref,
               wfc_ref, bfc_ref, out_ref, gx_ref, seq_ref, *, T, Bb, H):
    xv = x_ref[...].reshape(T * Bb, x_ref.shape[-1])
    gx_ref[...] = (
        jnp.dot(xv, wih0_ref[...], preferred_element_type=jnp.float32)
        + b0_ref[...]
    )

    for layer in range(2):
        whh = (whh0_ref if layer == 0 else whh1_ref)[...]
        h = jnp.zeros((Bb, H), jnp.float32)
        c = h
        for t in range(T):
            off = pl.multiple_of(t * Bb, Bb)
            gates = gx_ref[pl.ds(off, Bb), :] + jnp.dot(
                h.astype(jnp.bfloat16), whh, preferred_element_type=jnp.float32
            )
            s = jax.nn.sigmoid(gates)
            i_g = s[:, 0 * H:1 * H]
            f_g = s[:, 1 * H:2 * H]
            g_g = 2.0 * s[:, 2 * H:3 * H] - 1.0
            o_g = s[:, 3 * H:4 * H]
            c = f_g * c + i_g * g_g
            h = o_g * jnp.tanh(c)
            if layer == 0:
                seq_ref[pl.ds(off, Bb), :] = h.astype(jnp.bfloat16)
        if layer == 0:
            gx_ref[...] = (
                jnp.dot(seq_ref[...], wih1_ref[...],
                        preferred_element_type=jnp.float32)
                + b1_ref[...]
            )

    out_ref[...] = (
        jnp.dot(h.astype(jnp.bfloat16), wfc_ref[...],
                preferred_element_type=jnp.float32)
        + bfc_ref[...]
    )


def _scale_g(w, H):
    return jnp.concatenate(
        [w[..., :2 * H], 2.0 * w[..., 2 * H:3 * H], w[..., 3 * H:]], axis=-1
    )


@functools.partial(jax.jit, static_argnames=("block_b",))
def _forward(x, w_ih_0, w_hh_0, b_0, w_ih_1, w_hh_1, b_1, w_fc, b_fc,
             block_b=128):
    B, T, I = x.shape
    H = w_hh_0.shape[0]
    C = w_fc.shape[1]
    Bb = min(block_b, ((B + 7) // 8) * 8)
    Bp = ((B + Bb - 1) // Bb) * Bb

    xt = jnp.transpose(x, (1, 0, 2))
    if Bp != B:
        xt = jnp.pad(xt, ((0, 0), (0, Bp - B), (0, 0)))
    xt = xt.astype(jnp.bfloat16)

    bf = jnp.bfloat16
    args = [
        xt,
        _scale_g(w_ih_0, H).astype(bf), _scale_g(w_hh_0, H).astype(bf),
        _scale_g(b_0, H),
        _scale_g(w_ih_1, H).astype(bf), _scale_g(w_hh_1, H).astype(bf),
        _scale_g(b_1, H),
        w_fc.astype(bf), b_fc,
    ]

    body = functools.partial(_lstm_body, T=T, Bb=Bb, H=H)
    bcast = lambda shape: pl.BlockSpec(shape, lambda i: (0,) * len(shape))
    out = pl.pallas_call(
        body,
        out_shape=jax.ShapeDtypeStruct((Bp, C), jnp.float32),
        grid=(Bp // Bb,),
        in_specs=[
            pl.BlockSpec((T, Bb, I), lambda i: (0, i, 0)),
            bcast((I, 4 * H)), bcast((H, 4 * H)), bcast((1, 4 * H)),
            bcast((H, 4 * H)), bcast((H, 4 * H)), bcast((1, 4 * H)),
            bcast((H, C)), bcast((1, C)),
        ],
        out_specs=pl.BlockSpec((Bb, C), lambda i: (i, 0)),
        scratch_shapes=[
            pltpu.VMEM((T * Bb, 4 * H), jnp.float32),
            pltpu.VMEM((T * Bb, H), jnp.bfloat16),
        ],
        compiler_params=pltpu.CompilerParams(
            dimension_semantics=("parallel",),
        ),
    )(*args)
    return out[:B]


def kernel(x, w_ih_0, w_hh_0, b_0, w_ih_1, w_hh_1, b_1, w_fc, b_fc):
    return _forward(x, w_ih_0, w_hh_0, b_0, w_ih_1, w_hh_1, b_1, w_fc, b_fc)


# speedup vs baseline: 1.1496x; 1.1496x over previous
import functools

import jax
import jax.numpy as jnp
from jax.experimental import pallas as pl
from jax.experimental.pallas import tpu as pltpu


def _lstm_body(x_ref, wih0_ref, whh0_ref, b0_ref, wih1_ref, whh1_ref, b1_ref,
               wfc_ref, bfc_ref, out_ref, gx_ref, seq_ref, *, T, Bb, H):
    xv = x_ref[...].reshape(T * Bb, x_ref.shape[-1])
    gx_ref[...] = (
        jnp.dot(xv, wih0_ref[...], preferred_element_type=jnp.float32)
        + b0_ref[...]
    )

    for layer in range(2):
        whh = (whh0_ref if layer == 0 else whh1_ref)[...]
        h = jnp.zeros((Bb, H), jnp.float32)
        c = h
        for t in range(T):
            off = pl.multiple_of(t * Bb, Bb)
            gates = gx_ref[pl.ds(off, Bb), :] + jnp.dot(
                h.astype(jnp.bfloat16), whh, preferred_element_type=jnp.float32
            )
            s = jax.nn.sigmoid(gates)
            i_g = s[:, 0 * H:1 * H]
            f_g = s[:, 1 * H:2 * H]
            g_g = 2.0 * s[:, 2 * H:3 * H] - 1.0
            o_g = s[:, 3 * H:4 * H]
            c = f_g * c + i_g * g_g
            h = o_g * jnp.tanh(c)
            if layer == 0:
                seq_ref[pl.ds(off, Bb), :] = h.astype(jnp.bfloat16)
        if layer == 0:
            gx_ref[...] = (
                jnp.dot(seq_ref[...], wih1_ref[...],
                        preferred_element_type=jnp.float32)
                + b1_ref[...]
            )

    out_ref[...] = (
        jnp.dot(h.astype(jnp.bfloat16), wfc_ref[...],
                preferred_element_type=jnp.float32)
        + bfc_ref[...]
    )


def _scale_g(w, H):
    return jnp.concatenate(
        [w[..., :2 * H], 2.0 * w[..., 2 * H:3 * H], w[..., 3 * H:]], axis=-1
    )


@functools.partial(jax.jit, static_argnames=("block_b",))
def _forward(x, w_ih_0, w_hh_0, b_0, w_ih_1, w_hh_1, b_1, w_fc, b_fc,
             block_b=256):
    B, T, I = x.shape
    H = w_hh_0.shape[0]
    C = w_fc.shape[1]
    Bb = min(block_b, ((B + 7) // 8) * 8)
    Bp = ((B + Bb - 1) // Bb) * Bb

    xt = jnp.transpose(x, (1, 0, 2))
    if Bp != B:
        xt = jnp.pad(xt, ((0, 0), (0, Bp - B), (0, 0)))
    xt = xt.astype(jnp.bfloat16)

    bf = jnp.bfloat16
    args = [
        xt,
        _scale_g(w_ih_0, H).astype(bf), _scale_g(w_hh_0, H).astype(bf),
        _scale_g(b_0, H),
        _scale_g(w_ih_1, H).astype(bf), _scale_g(w_hh_1, H).astype(bf),
        _scale_g(b_1, H),
        w_fc.astype(bf), b_fc,
    ]

    body = functools.partial(_lstm_body, T=T, Bb=Bb, H=H)
    bcast = lambda shape: pl.BlockSpec(shape, lambda i: (0,) * len(shape))
    out = pl.pallas_call(
        body,
        out_shape=jax.ShapeDtypeStruct((Bp, C), jnp.float32),
        grid=(Bp // Bb,),
        in_specs=[
            pl.BlockSpec((T, Bb, I), lambda i: (0, i, 0)),
            bcast((I, 4 * H)), bcast((H, 4 * H)), bcast((1, 4 * H)),
            bcast((H, 4 * H)), bcast((H, 4 * H)), bcast((1, 4 * H)),
            bcast((H, C)), bcast((1, C)),
        ],
        out_specs=pl.BlockSpec((Bb, C), lambda i: (i, 0)),
        scratch_shapes=[
            pltpu.VMEM((T * Bb, 4 * H), jnp.float32),
            pltpu.VMEM((T * Bb, H), jnp.bfloat16),
        ],
        compiler_params=pltpu.CompilerParams(
            dimension_semantics=("parallel",),
        ),
    )(*args)
    return out[:B]


def kernel(x, w_ih_0, w_hh_0, b_0, w_ih_1, w_hh_1, b_1, w_fc, b_fc):
    return _forward(x, w_ih_0, w_hh_0, b_0, w_ih_1, w_hh_1, b_1, w_fc, b_fc)
